# baseline (speedup 1.0000x reference)
import jax
import jax.numpy as jnp
from jax import lax
from jax.experimental import pallas as pl
from jax.experimental.pallas import tpu as pltpu
from jax.experimental.pallas import tpu_sc as plsc

ATTR_DIM = 26
N_OBJ = 20
EMBED = 64
BS = 1024
POS = ATTR_DIM * N_OBJ
ROWS = BS * POS
NC = 2
NS = 16
NW = NC * NS
BPW = BS // NW
CHUNK = 40
CPB = POS // CHUNK
NCH = BPW * CPB
OUT_F = CHUNK * 2 * EMBED
NBUF = 8


def _sc_body(cond_hbm, emb_hbm, out_hbm, embv, *bufs):
    c = lax.axis_index("c")
    s = lax.axis_index("s")
    wid = s * NC + c
    b_base = wid * BPW

    xps = bufs[0:NBUF]
    xns = bufs[NBUF:2 * NBUF]
    obs = bufs[2 * NBUF:3 * NBUF]
    sxps = bufs[3 * NBUF:4 * NBUF]
    sxns = bufs[4 * NBUF:5 * NBUF]
    sos = bufs[5 * NBUF:6 * NBUF]

    pltpu.sync_copy(emb_hbm, embv)

    def prefill(ob):
        def per(r, carry):
            er = r * EMBED
            ob_off = r * 128 + 64
            for j in range(4):
                ob[pl.ds(ob_off + j * 16, 16)] = embv[pl.ds(er + j * 16, 16)]
            return carry
        lax.fori_loop(0, N_OBJ, per, 0)

        def cp(r, carry):
            o = r * 128 + 64
            for j in range(4):
                ob[pl.ds(N_OBJ * 128 + o + j * 16, 16)] = ob[pl.ds(o + j * 16, 16)]
            return carry
        lax.fori_loop(0, N_OBJ, cp, 0)

    for ob in obs:
        prefill(ob)

    def chunk_coords(chunk):
        be = chunk // CPB
        p0 = (chunk - be * CPB) * CHUNK
        return b_base + be, p0

    def gather(chunk, b):
        bi, p0 = chunk_coords(chunk)
        pltpu.async_copy(cond_hbm.at[0, bi, pl.ds(p0, CHUNK), :], xps[b], sxps[b])
        pltpu.async_copy(cond_hbm.at[1, bi, pl.ds(p0, CHUNK), :], xns[b], sxns[b])

    for b in range(NBUF):
        gather(b, b)

    def outer(g, carry):
        for b in range(NBUF):
            chunk = g * NBUF + b
            bi, p0 = chunk_coords(chunk)

            @pl.when(g >= 1)
            def _wait_prev_scatter():
                pbi, pp0 = chunk_coords(chunk - NBUF)
                poff = (pbi * POS + pp0) * 128
                pltpu.make_async_copy(
                    obs[b], out_hbm.at[pl.ds(poff, OUT_F)], sos[b]).wait()

            pltpu.make_async_copy(
                cond_hbm.at[0, bi, pl.ds(p0, CHUNK), :], xps[b], sxps[b]).wait()
            pltpu.make_async_copy(
                cond_hbm.at[1, bi, pl.ds(p0, CHUNK), :], xns[b], sxns[b]).wait()

            xpb, xnb, obb = xps[b], xns[b], obs[b]

            def comp(r, carry2):
                r2 = r * 2
                xs = [xpb[r2 + (j // 4), pl.ds((j % 4) * 16, 16)] for j in range(8)]
                ys = [xnb[r2 + (j // 4), pl.ds((j % 4) * 16, 16)] for j in range(8)]
                zs = [x + y for x, y in zip(xs, ys)]
                o0 = r * 256
                for j in range(4):
                    obb[pl.ds(o0 + j * 16, 16)] = zs[j]
                for j in range(4):
                    obb[pl.ds(o0 + 128 + j * 16, 16)] = zs[4 + j]
                return carry2
            lax.fori_loop(0, CHUNK // 2, comp, 0)

            off = (bi * POS + p0) * 128
            pltpu.async_copy(obb, out_hbm.at[pl.ds(off, OUT_F)], sos[b])

            @pl.when(g < NCH // NBUF - 1)
            def _prefetch():
                gather(chunk + NBUF, b)
        return carry

    lax.fori_loop(0, NCH // NBUF, outer, 0)

    for b in range(NBUF):
        chunk = NCH - NBUF + b
        bi, p0 = chunk_coords(chunk)
        off = (bi * POS + p0) * 128
        pltpu.make_async_copy(
            obs[b], out_hbm.at[pl.ds(off, OUT_F)], sos[b]).wait()


def kernel(cond, emb):
    emb_flat = emb.reshape(N_OBJ * EMBED)
    vm_in = pltpu.VMEM((CHUNK, EMBED), jnp.float32)
    vm_out = pltpu.VMEM((OUT_F,), jnp.float32)
    run = pl.kernel(
        _sc_body,
        out_type=jax.ShapeDtypeStruct((ROWS * 2 * EMBED,), jnp.float32),
        mesh=plsc.VectorSubcoreMesh(
            core_axis_name="c", subcore_axis_name="s",
            num_cores=NC, num_subcores=NS),
        scratch_types=(
            [pltpu.VMEM((N_OBJ * EMBED,), jnp.float32)]
            + [vm_in] * (2 * NBUF) + [vm_out] * NBUF
            + [pltpu.SemaphoreType.DMA] * (3 * NBUF)
        ),
    )
    out = run(cond, emb_flat)
    return out.reshape(BS, POS, 2 * EMBED)


# speedup vs baseline: 1.0048x; 1.0048x over previous
import jax
import jax.numpy as jnp
from jax import lax
from jax.experimental import pallas as pl
from jax.experimental.pallas import tpu as pltpu
from jax.experimental.pallas import tpu_sc as plsc

ATTR_DIM = 26
N_OBJ = 20
EMBED = 64
BS = 1024
POS = ATTR_DIM * N_OBJ
ROWS = BS * POS
NC = 2
NS = 16
NW = NC * NS
BPW = BS // NW
CHUNK = 104
CPB = POS // CHUNK
NCH = BPW * CPB
OUT_F = CHUNK * 2 * EMBED
NBUF = 2
PATR = CHUNK + N_OBJ


def _sc_body(cond_hbm, emb_hbm, out_hbm, embv, patv, *bufs):
    c = lax.axis_index("c")
    s = lax.axis_index("s")
    wid = s * NC + c
    b_base = wid * BPW

    xps = bufs[0:NBUF]
    xns = bufs[NBUF:2 * NBUF]
    obs = bufs[2 * NBUF:3 * NBUF]
    sxps = bufs[3 * NBUF:4 * NBUF]
    sxns = bufs[4 * NBUF:5 * NBUF]
    sos = bufs[5 * NBUF:6 * NBUF]

    pltpu.sync_copy(emb_hbm, embv)

    def pat(k, carry):
        er = lax.rem(k, N_OBJ) * EMBED
        for j in range(4):
            patv[pl.ds(k * EMBED + j * 16, 16)] = embv[pl.ds(er + j * 16, 16)]
        return carry
    lax.fori_loop(0, PATR, pat, 0)

    def chunk_coords(chunk):
        be = chunk // CPB
        p0 = (chunk - be * CPB) * CHUNK
        return b_base + be, p0

    def gather(chunk, b):
        bi, p0 = chunk_coords(chunk)
        pltpu.async_copy(cond_hbm.at[0, bi, pl.ds(p0, CHUNK), :], xps[b], sxps[b])
        pltpu.async_copy(cond_hbm.at[1, bi, pl.ds(p0, CHUNK), :], xns[b], sxns[b])

    for b in range(NBUF):
        gather(b, b)

    def outer(g, carry):
        for b in range(NBUF):
            chunk = g * NBUF + b
            bi, p0 = chunk_coords(chunk)

            @pl.when(g >= 1)
            def _wait_prev_scatter():
                pbi, pp0 = chunk_coords(chunk - NBUF)
                poff = (pbi * POS + pp0) * 128
                pltpu.make_async_copy(
                    obs[b], out_hbm.at[pl.ds(poff, OUT_F)], sos[b]).wait()

            pltpu.make_async_copy(
                cond_hbm.at[0, bi, pl.ds(p0, CHUNK), :], xps[b], sxps[b]).wait()
            pltpu.make_async_copy(
                cond_hbm.at[1, bi, pl.ds(p0, CHUNK), :], xns[b], sxns[b]).wait()

            xpb, xnb, obb = xps[b], xns[b], obs[b]
            rho = lax.rem(p0, N_OBJ)

            def comp(r, carry2):
                r2 = r * 2
                xs = [xpb[r2 + (j // 4), pl.ds((j % 4) * 16, 16)] for j in range(8)]
                ys = [xnb[r2 + (j // 4), pl.ds((j % 4) * 16, 16)] for j in range(8)]
                es = [patv[pl.ds((rho + r2 + (j // 4)) * EMBED + (j % 4) * 16, 16)]
                      for j in range(8)]
                zs = [x + y for x, y in zip(xs, ys)]
                o0 = r * 256
                for j in range(4):
                    obb[pl.ds(o0 + j * 16, 16)] = zs[j]
                for j in range(4):
                    obb[pl.ds(o0 + 64 + j * 16, 16)] = es[j]
                for j in range(4):
                    obb[pl.ds(o0 + 128 + j * 16, 16)] = zs[4 + j]
                for j in range(4):
                    obb[pl.ds(o0 + 192 + j * 16, 16)] = es[4 + j]
                return carry2
            lax.fori_loop(0, CHUNK // 2, comp, 0)

            off = (bi * POS + p0) * 128
            pltpu.async_copy(obb, out_hbm.at[pl.ds(off, OUT_F)], sos[b])

            @pl.when(g < NCH // NBUF - 1)
            def _prefetch():
                gather(chunk + NBUF, b)
        return carry

    lax.fori_loop(0, NCH // NBUF, outer, 0)

    for b in range(NBUF):
        chunk = NCH - NBUF + b
        bi, p0 = chunk_coords(chunk)
        off = (bi * POS + p0) * 128
        pltpu.make_async_copy(
            obs[b], out_hbm.at[pl.ds(off, OUT_F)], sos[b]).wait()


def kernel(cond, emb):
    emb_flat = emb.reshape(N_OBJ * EMBED)
    vm_in = pltpu.VMEM((CHUNK, EMBED), jnp.float32)
    vm_out = pltpu.VMEM((OUT_F,), jnp.float32)
    run = pl.kernel(
        _sc_body,
        out_type=jax.ShapeDtypeStruct((ROWS * 2 * EMBED,), jnp.float32),
        mesh=plsc.VectorSubcoreMesh(
            core_axis_name="c", subcore_axis_name="s",
            num_cores=NC, num_subcores=NS),
        scratch_types=(
            [pltpu.VMEM((N_OBJ * EMBED,), jnp.float32),
             pltpu.VMEM((PATR * EMBED,), jnp.float32)]
            + [vm_in] * (2 * NBUF) + [vm_out] * NBUF
            + [pltpu.SemaphoreType.DMA] * (3 * NBUF)
        ),
    )
    out = run(cond, emb_flat)
    return out.reshape(BS, POS, 2 * EMBED)
